# Initial kernel scaffold; baseline (speedup 1.0000x reference)
import jax
import jax.numpy as jnp
from jax.experimental import pallas as pl


def kernel(preds, targets):
    raise NotImplementedError("write your pallas kernel here")


# speedup vs baseline: 7.3106x; 7.3106x over previous
import functools

import jax
import jax.numpy as jnp
from jax.experimental import pallas as pl
from jax.experimental.pallas import tpu as pltpu

MARGIN = 1.0
PADD_IDX = 0

_B = 256


def _margin_body(tcol_blk_ref, tcol_full_ref, preds_ref, out_ref,
                 w_ref, tot_ref, cnt_ref):
    b = pl.program_id(0)
    k = pl.program_id(1)
    nb = pl.num_programs(0)
    nk = pl.num_programs(1)
    V = preds_ref.shape[-1]

    @pl.when(jnp.logical_and(b == 0, k == 0))
    def _init():
        acc = jnp.zeros((1, V), jnp.float32)
        n_full = tcol_full_ref.shape[0]
        chunk = 512
        iota_v = jax.lax.broadcasted_iota(jnp.int32, (chunk, V), 1)
        for c in range(n_full // chunk):
            tc = tcol_full_ref[c * chunk:(c + 1) * chunk, :]
            onehot = (tc == iota_v).astype(jnp.float32)
            acc = acc + jnp.sum(onehot, axis=0, keepdims=True)
        col = jax.lax.broadcasted_iota(jnp.int32, (1, V), 1)
        w_ref[...] = jnp.where(col == PADD_IDX, 0.0, acc)
        tot_ref[0, 0] = 0.0
        cnt_ref[0, 0] = 0.0

    rows = preds_ref[0]
    t_blk = tcol_blk_ref[...]
    iota_bv = jax.lax.broadcasted_iota(jnp.int32, rows.shape, 1)
    d = jnp.sum(jnp.where(iota_bv == t_blk, rows, 0.0), axis=1, keepdims=True)
    relu = jnp.maximum(rows + (MARGIN - d), 0.0)
    row_sums = jnp.sum(relu * w_ref[...], axis=1, keepdims=True)
    mask = (t_blk != PADD_IDX).astype(jnp.float32)
    tot_ref[0, 0] += jnp.sum(row_sums * mask)
    cnt_ref[0, 0] += jnp.sum(mask)

    @pl.when(jnp.logical_and(b == nb - 1, k == nk - 1))
    def _fini():
        cnt = cnt_ref[0, 0]
        out_ref[...] = jnp.full((1, 1), tot_ref[0, 0] / (cnt * cnt),
                                jnp.float32)


def kernel(preds, targets):
    Bt, T1, V = preds.shape
    T = T1 - 1
    N = Bt * T
    t32 = targets.astype(jnp.int32)
    tcol = t32.reshape(N, 1)

    nk = T // _B
    out = pl.pallas_call(
        _margin_body,
        grid=(Bt, nk),
        in_specs=[
            pl.BlockSpec((_B, 1), lambda b, k: (b * nk + k, b - b)),
            pl.BlockSpec((N, 1), lambda b, k: (b - b, k - k)),
            pl.BlockSpec((1, _B, V), lambda b, k: (b, k, b - b)),
        ],
        out_specs=pl.BlockSpec((1, 1), lambda b, k: (b - b, k - k)),
        out_shape=jax.ShapeDtypeStruct((1, 1), jnp.float32),
        scratch_shapes=[
            pltpu.VMEM((1, V), jnp.float32),
            pltpu.SMEM((1, 1), jnp.float32),
            pltpu.SMEM((1, 1), jnp.float32),
        ],
        compiler_params=pltpu.CompilerParams(
            dimension_semantics=("arbitrary", "arbitrary")),
    )(tcol, tcol, preds)
    return out.reshape(())


# speedup vs baseline: 7.3167x; 1.0008x over previous
import functools

import jax
import jax.numpy as jnp
from jax.experimental import pallas as pl
from jax.experimental.pallas import tpu as pltpu

MARGIN = 1.0
PADD_IDX = 0

_B = 256


def _margin_body(tcol_blk_ref, tcol_full_ref, preds_ref, out_ref,
                 w_ref, tot_ref, cnt_ref):
    b = pl.program_id(0)
    k = pl.program_id(1)
    nb = pl.num_programs(0)
    nk = pl.num_programs(1)
    V = preds_ref.shape[-1]

    @pl.when(jnp.logical_and(b == 0, k == 0))
    def _init():
        acc = jnp.zeros((1, V), jnp.float32)
        n_full = tcol_full_ref.shape[0]
        chunk = 512
        iota_v = jax.lax.broadcasted_iota(jnp.int32, (chunk, V), 1)
        for c in range(n_full // chunk):
            tc = tcol_full_ref[c * chunk:(c + 1) * chunk, :]
            onehot = (tc == iota_v).astype(jnp.float32)
            acc = acc + jnp.sum(onehot, axis=0, keepdims=True)
        col = jax.lax.broadcasted_iota(jnp.int32, (1, V), 1)
        w_ref[...] = jnp.where(col == PADD_IDX, 0.0, acc)
        tot_ref[0, 0] = 0.0
        cnt_ref[0, 0] = 0.0

    rows = preds_ref[0]
    t_blk = tcol_blk_ref[...]
    iota_bv = jax.lax.broadcasted_iota(jnp.int32, rows.shape, 1)
    d = jnp.sum(jnp.where(iota_bv == t_blk, rows, 0.0), axis=1, keepdims=True)
    relu = jnp.maximum(rows + (MARGIN - d), 0.0)
    row_sums = jax.lax.dot_general(
        relu, w_ref[...], (((1,), (1,)), ((), ())),
        preferred_element_type=jnp.float32,
        precision=jax.lax.Precision.HIGHEST)
    mask = (t_blk != PADD_IDX).astype(jnp.float32)
    tot_ref[0, 0] += jnp.sum(row_sums * mask)
    cnt_ref[0, 0] += jnp.sum(mask)

    @pl.when(jnp.logical_and(b == nb - 1, k == nk - 1))
    def _fini():
        cnt = cnt_ref[0, 0]
        out_ref[...] = jnp.full((1, 1), tot_ref[0, 0] / (cnt * cnt),
                                jnp.float32)


def kernel(preds, targets):
    Bt, T1, V = preds.shape
    T = T1 - 1
    N = Bt * T
    t32 = targets.astype(jnp.int32)
    tcol = t32.reshape(N, 1)

    nk = T // _B
    out = pl.pallas_call(
        _margin_body,
        grid=(Bt, nk),
        in_specs=[
            pl.BlockSpec((_B, 1), lambda b, k: (b * nk + k, b - b)),
            pl.BlockSpec((N, 1), lambda b, k: (b - b, k - k)),
            pl.BlockSpec((1, _B, V), lambda b, k: (b, k, b - b)),
        ],
        out_specs=pl.BlockSpec((1, 1), lambda b, k: (b - b, k - k)),
        out_shape=jax.ShapeDtypeStruct((1, 1), jnp.float32),
        scratch_shapes=[
            pltpu.VMEM((1, V), jnp.float32),
            pltpu.SMEM((1, 1), jnp.float32),
            pltpu.SMEM((1, 1), jnp.float32),
        ],
        compiler_params=pltpu.CompilerParams(
            dimension_semantics=("arbitrary", "arbitrary")),
    )(tcol, tcol, preds)
    return out.reshape(())
